# Initial kernel scaffold; baseline (speedup 1.0000x reference)
import jax
import jax.numpy as jnp
from jax.experimental import pallas as pl


def kernel(i, p0, p1, center_offset):
    raise NotImplementedError("write your pallas kernel here")


# speedup vs baseline: 7.9094x; 7.9094x over previous
import functools

import jax
import jax.numpy as jnp
from jax import lax
from jax.experimental import pallas as pl
from jax.experimental.pallas import tpu as pltpu
from jax.experimental.pallas import tpu_sc as plsc

N = 256
B = 4
C = 128
TBL = 64
NC, NS = 2, 16
NW = NC * NS
ROWS = N * N * B
RPW = ROWS // NW
CHUNK = 128
NCH = RPW // CHUNK


def _prep_kernel(i_ref, p0_ref, p1_ref, co_ref, table_ref, idx_ref, cm_ref):
    zrow = jnp.zeros((1, C), jnp.float32)
    p0p = jnp.concatenate([p0_ref[...], zrow], axis=0)
    p1p = jnp.concatenate([p1_ref[...], zrow], axis=0)
    table_ref[...] = p0p[:, None, :] + p1p[None, :, :]

    co0 = co_ref[0]
    co1 = co_ref[1]
    for bat in range(B):
        i0 = i_ref[:, bat, 0]
        i1 = i_ref[:, bat, 1]
        r0 = i0[:, None] - i0[None, :] + co0
        r1 = i1[:, None] - i1[None, :] + co1
        cm_ref[bat] = jnp.where((r0 < 0) | (r1 < 0),
                                jnp.int32(1), jnp.int32(0))
        idx_ref[bat] = jnp.maximum(r0, 0) * TBL + jnp.maximum(r1, 0)


def _prep(i, p0, p1, center_offset):
    return pl.pallas_call(
        _prep_kernel,
        in_specs=[
            pl.BlockSpec(memory_space=pltpu.VMEM),
            pl.BlockSpec(memory_space=pltpu.VMEM),
            pl.BlockSpec(memory_space=pltpu.VMEM),
            pl.BlockSpec(memory_space=pltpu.SMEM),
        ],
        out_specs=[
            pl.BlockSpec(memory_space=pltpu.VMEM),
            pl.BlockSpec(memory_space=pltpu.VMEM),
            pl.BlockSpec(memory_space=pltpu.VMEM),
        ],
        out_shape=[
            jax.ShapeDtypeStruct((TBL, TBL, C), jnp.float32),
            jax.ShapeDtypeStruct((B, N, N), jnp.int32),
            jax.ShapeDtypeStruct((B, N, N), jnp.int32),
        ],
    )(i, p0, p1, center_offset)


def _gather_body(table_hbm, idx_hbm, out_hbm, idx_v, rows_v, gsem):
    wid = lax.axis_index("s") * NC + lax.axis_index("c")
    pltpu.sync_copy(idx_hbm.at[wid], idx_v)
    base = wid * RPW

    def body(j, _):
        pltpu.async_copy(table_hbm.at[idx_v.at[j]], rows_v, gsem).wait()
        pltpu.sync_copy(rows_v, out_hbm.at[pl.ds(base + j * CHUNK, CHUNK)])
        return 0

    lax.fori_loop(0, NCH, body, 0)


@functools.partial(
    pl.kernel,
    mesh=plsc.VectorSubcoreMesh(core_axis_name="c", subcore_axis_name="s"),
    out_type=jax.ShapeDtypeStruct((ROWS, C), jnp.float32),
    scratch_types=[
        pltpu.VMEM((NCH, CHUNK), jnp.int32),
        pltpu.VMEM((CHUNK, C), jnp.float32),
        pltpu.SemaphoreType.DMA,
    ],
)
def _sc_gather(table_hbm, idx_hbm, out_hbm, idx_v, rows_v, gsem):
    _gather_body(table_hbm, idx_hbm, out_hbm, idx_v, rows_v, gsem)


def kernel(i, p0, p1, center_offset):
    i = i.astype(jnp.int32)
    center_offset = center_offset.astype(jnp.int32)
    table, idx, cm = _prep(i, p0, p1, center_offset)
    table = table.reshape(TBL * TBL, C)
    idx = jnp.transpose(idx, (1, 2, 0)).reshape(NW, NCH, CHUNK)
    pe = _sc_gather(table, idx)
    pe = pe.reshape(N, N, B, C)
    cm = jnp.transpose(cm, (1, 2, 0)).astype(jnp.bool_)
    return pe, cm
